# baseline (speedup 1.0000x reference)
import functools

import jax
import jax.numpy as jnp
from jax import lax
from jax.experimental import pallas as pl
from jax.experimental.pallas import tpu as pltpu
from jax.experimental.pallas import tpu_sc as plsc

THRESH_IGNORE = 255
K_KEEP = 100000

B, C, H, W = 8, 19, 512, 512
N = B * H * W
SPLIT = 4
N0 = SPLIT * H * W
N1 = N - N0

SHIFT = 21
NBINS = 1024
NLANE = 16
TROWS = NBINS * NLANE // 128

NWORKERS = 32

ROWS = 256
OROWS = ROWS * W // 128


def _loss_body(x_ref, t_ref, out_ref):
    x = x_ref[0]
    t = t_ref[0]
    m = jnp.max(x, axis=0)
    s = jnp.sum(jnp.exp(x - m[None]), axis=0)
    xt = jnp.zeros_like(m)
    for c in range(C):
        xt = jnp.where(t == c, x[c], xt)
    nll = jnp.log(s) + m - xt
    nll = jnp.where(t == THRESH_IGNORE, 0.0, nll)
    for j in range(W // 128):
        out_ref[pl.ds(j * ROWS, ROWS), :] = nll[:, j * 128:(j + 1) * 128]


def _pixel_losses(inputs, targets, b0, bn):
    grid = (bn, H // ROWS)
    nb = H // ROWS

    return pl.pallas_call(
        _loss_body,
        grid=grid,
        in_specs=[
            pl.BlockSpec((1, C, ROWS, W), lambda b, h: (b + b0, 0, h, 0)),
            pl.BlockSpec((1, ROWS, W), lambda b, h: (b + b0, h, 0)),
        ],
        out_specs=pl.BlockSpec((OROWS, 128), lambda b, h: (b * nb + h, 0)),
        out_shape=jax.ShapeDtypeStruct((bn * nb * OROWS, 128), jnp.float32),
    )(inputs, targets)


def _make_sc_histograms(n_elems):
    chunk = n_elems // NWORKERS
    npiece = 2
    piece = chunk // npiece

    def _sc_hist_body(loss_hbm, cnt_out, sum_out,
                      buf0, buf1, cnt_tab, sum_tab, sem0, sem1):
        cid = lax.axis_index("c")
        sid = lax.axis_index("s")
        wid = sid * 2 + cid
        base = wid * chunk

        zeros = jnp.zeros((NLANE,), jnp.float32)

        @plsc.parallel_loop(0, TROWS * 8, unroll=8)
        def zero_body(i):
            r = lax.shift_right_logical(i, 3)
            c0 = (i & 7) * NLANE
            cnt_tab[r, pl.ds(c0, NLANE)] = zeros
            sum_tab[r, pl.ds(c0, NLANE)] = zeros

        lane = lax.iota(jnp.int32, NLANE)
        ones = jnp.ones((NLANE,), jnp.float32)
        bufs = (buf0, buf1)
        sems = (sem0, sem1)

        copies = [None] * npiece
        copies[0] = pltpu.make_async_copy(
            loss_hbm.at[pl.ds(base, piece)], buf0, sem0)
        copies[0].start()
        for p in range(npiece):
            if p + 1 < npiece:
                copies[p + 1] = pltpu.make_async_copy(
                    loss_hbm.at[pl.ds(base + (p + 1) * piece, piece)],
                    bufs[(p + 1) % 2], sems[(p + 1) % 2])
                copies[p + 1].start()
            copies[p].wait()
            buf = bufs[p % 2]

            @plsc.parallel_loop(0, piece // NLANE, unroll=8)
            def hist_body(i):
                v = buf[pl.ds(i * NLANE, NLANE)]
                bits = plsc.bitcast(v, jnp.int32)
                idx = lax.shift_right_logical(bits, SHIFT) * NLANE + lane
                row = lax.shift_right_logical(idx, 7)
                col = idx & 127
                plsc.addupdate_scatter(cnt_tab, [row, col], ones)
                plsc.addupdate_scatter(sum_tab, [row, col], v)

        pltpu.sync_copy(cnt_tab, cnt_out.at[wid])
        pltpu.sync_copy(sum_tab, sum_out.at[wid])

    mesh = plsc.VectorSubcoreMesh(core_axis_name="c", subcore_axis_name="s")
    return functools.partial(
        pl.kernel,
        mesh=mesh,
        compiler_params=pltpu.CompilerParams(needs_layout_passes=False),
        out_type=(
            jax.ShapeDtypeStruct((NWORKERS, TROWS, 128), jnp.float32),
            jax.ShapeDtypeStruct((NWORKERS, TROWS, 128), jnp.float32),
        ),
        scratch_types=[
            pltpu.VMEM((piece,), jnp.float32),
            pltpu.VMEM((piece,), jnp.float32),
            pltpu.VMEM((TROWS, 128), jnp.float32),
            pltpu.VMEM((TROWS, 128), jnp.float32),
            pltpu.SemaphoreType.DMA,
            pltpu.SemaphoreType.DMA,
        ],
    )(_sc_hist_body)


def _select_body(cnt0_ref, sum0_ref, cnt1_ref, sum1_ref, out_ref):
    cnt = jnp.sum(cnt0_ref[...], axis=0) + jnp.sum(cnt1_ref[...], axis=0)
    sm = jnp.sum(sum0_ref[...], axis=0) + jnp.sum(sum1_ref[...], axis=0)

    r_i = lax.broadcasted_iota(jnp.int32, (TROWS, 128), 0)
    c_i = lax.broadcasted_iota(jnp.int32, (TROWS, 128), 1)
    bin_idx = lax.shift_right_logical(r_i * 128 + c_i, 4)

    kf = jnp.float32(K_KEEP)

    def n_gt(x):
        return jnp.sum(jnp.where(bin_idx > x, cnt, 0.0))

    def bis(_, lohi):
        lo, hi = lohi
        mid = (lo + hi) // 2
        below = n_gt(mid) < kf
        return jnp.where(below, lo, mid), jnp.where(below, mid, hi)

    lo, hi = lax.fori_loop(0, 12, bis, (jnp.int32(-1), jnp.int32(NBINS - 1)))
    b_sel = hi

    count_above = n_gt(b_sel)
    sum_above = jnp.sum(jnp.where(bin_idx > b_sel, sm, 0.0))
    c_b = jnp.sum(jnp.where(bin_idx == b_sel, cnt, 0.0))
    s_b = jnp.sum(jnp.where(bin_idx == b_sel, sm, 0.0))
    r = kf - count_above

    lo_f_vec = lax.bitcast_convert_type(
        lax.shift_left(bin_idx, SHIFT), jnp.float32)
    lo_f = jnp.sum(jnp.where(bin_idx == b_sel, lo_f_vec, 0.0)) / NLANE
    hi_f = jnp.sum(jnp.where(bin_idx == b_sel + 1, lo_f_vec, 0.0)) / NLANE
    width = hi_f - lo_f

    mhat = (s_b / c_b - lo_f) / width
    bq = jnp.clip(12.0 * mhat - 6.0, -2.0, 2.0)
    aq = 1.0 - bq / 2.0
    tq = r / c_b
    small_b = jnp.abs(bq) < 1e-3
    b_safe = jnp.where(small_b, 1.0, bq)
    disc = jnp.maximum(aq * aq + 2.0 * b_safe * (aq + b_safe / 2.0 - tq), 0.0)
    sdisc = jnp.max(jnp.sqrt(jnp.full((8, 128), disc, jnp.float32)))
    q_quad = (sdisc - aq) / b_safe
    q_lin = 1.0 - tq / aq
    q = jnp.clip(jnp.where(small_b, q_lin, q_quad), 0.0, 1.0)
    iu = aq * (1.0 - q * q) / 2.0 + bq * (1.0 - q * q * q) / 3.0
    top_r = c_b * width * iu + r * lo_f
    out_ref[0, 0] = (sum_above + top_r) / kf


def _select(cnt0, sm0, cnt1, sm1):
    spec = pl.BlockSpec((NWORKERS, TROWS, 128), lambda: (0, 0, 0))
    return pl.pallas_call(
        _select_body,
        in_specs=[spec, spec, spec, spec],
        out_specs=pl.BlockSpec(memory_space=pltpu.SMEM),
        out_shape=jax.ShapeDtypeStruct((1, 1), jnp.float32),
    )(cnt0, sm0, cnt1, sm1)


def kernel(inputs, targets):
    sc_hist = _make_sc_histograms(N0)
    losses0 = _pixel_losses(inputs, targets, 0, SPLIT)
    cnt0, sm0 = sc_hist(losses0.reshape(N0))
    losses1 = _pixel_losses(inputs, targets, SPLIT, B - SPLIT)
    if N1 == N0:
        cnt1, sm1 = sc_hist(losses1.reshape(N1))
    else:
        cnt1, sm1 = _make_sc_histograms(N1)(losses1.reshape(N1))
    out = _select(cnt0, sm0, cnt1, sm1)
    return out[0, 0]


# speedup vs baseline: 1.0177x; 1.0177x over previous
import functools

import jax
import jax.numpy as jnp
from jax import lax
from jax.experimental import pallas as pl
from jax.experimental.pallas import tpu as pltpu
from jax.experimental.pallas import tpu_sc as plsc

THRESH_IGNORE = 255
K_KEEP = 100000

B, C, H, W = 8, 19, 512, 512
N = B * H * W
SPLIT = 6
N0 = SPLIT * H * W
N1 = N - N0

SHIFT = 21
NBINS = 1024
NLANE = 16
TROWS = NBINS * NLANE // 128

NWORKERS = 32

ROWS = 256
OROWS = ROWS * W // 128


def _loss_body(x_ref, t_ref, out_ref):
    x = x_ref[0]
    t = t_ref[0]
    m = jnp.max(x, axis=0)
    s = jnp.sum(jnp.exp(x - m[None]), axis=0)
    xt = jnp.zeros_like(m)
    for c in range(C):
        xt = jnp.where(t == c, x[c], xt)
    nll = jnp.log(s) + m - xt
    nll = jnp.where(t == THRESH_IGNORE, 0.0, nll)
    for j in range(W // 128):
        out_ref[pl.ds(j * ROWS, ROWS), :] = nll[:, j * 128:(j + 1) * 128]


def _pixel_losses(inputs, targets, b0, bn):
    grid = (bn, H // ROWS)
    nb = H // ROWS

    return pl.pallas_call(
        _loss_body,
        grid=grid,
        in_specs=[
            pl.BlockSpec((1, C, ROWS, W), lambda b, h: (b + b0, 0, h, 0)),
            pl.BlockSpec((1, ROWS, W), lambda b, h: (b + b0, h, 0)),
        ],
        out_specs=pl.BlockSpec((OROWS, 128), lambda b, h: (b * nb + h, 0)),
        out_shape=jax.ShapeDtypeStruct((bn * nb * OROWS, 128), jnp.float32),
    )(inputs, targets)


def _make_sc_histograms(n_elems):
    chunk = n_elems // NWORKERS
    npiece = 2
    piece = chunk // npiece

    def _sc_hist_body(loss_hbm, cnt_out, sum_out,
                      buf0, buf1, cnt_tab, sum_tab, sem0, sem1):
        cid = lax.axis_index("c")
        sid = lax.axis_index("s")
        wid = sid * 2 + cid
        base = wid * chunk

        zeros = jnp.zeros((NLANE,), jnp.float32)

        @plsc.parallel_loop(0, TROWS * 8, unroll=8)
        def zero_body(i):
            r = lax.shift_right_logical(i, 3)
            c0 = (i & 7) * NLANE
            cnt_tab[r, pl.ds(c0, NLANE)] = zeros
            sum_tab[r, pl.ds(c0, NLANE)] = zeros

        lane = lax.iota(jnp.int32, NLANE)
        ones = jnp.ones((NLANE,), jnp.float32)
        bufs = (buf0, buf1)
        sems = (sem0, sem1)

        copies = [None] * npiece
        copies[0] = pltpu.make_async_copy(
            loss_hbm.at[pl.ds(base, piece)], buf0, sem0)
        copies[0].start()
        for p in range(npiece):
            if p + 1 < npiece:
                copies[p + 1] = pltpu.make_async_copy(
                    loss_hbm.at[pl.ds(base + (p + 1) * piece, piece)],
                    bufs[(p + 1) % 2], sems[(p + 1) % 2])
                copies[p + 1].start()
            copies[p].wait()
            buf = bufs[p % 2]

            @plsc.parallel_loop(0, piece // NLANE, unroll=8)
            def hist_body(i):
                v = buf[pl.ds(i * NLANE, NLANE)]
                bits = plsc.bitcast(v, jnp.int32)
                idx = lax.shift_right_logical(bits, SHIFT) * NLANE + lane
                row = lax.shift_right_logical(idx, 7)
                col = idx & 127
                plsc.addupdate_scatter(cnt_tab, [row, col], ones)
                plsc.addupdate_scatter(sum_tab, [row, col], v)

        pltpu.sync_copy(cnt_tab, cnt_out.at[wid])
        pltpu.sync_copy(sum_tab, sum_out.at[wid])

    mesh = plsc.VectorSubcoreMesh(core_axis_name="c", subcore_axis_name="s")
    return functools.partial(
        pl.kernel,
        mesh=mesh,
        compiler_params=pltpu.CompilerParams(needs_layout_passes=False),
        out_type=(
            jax.ShapeDtypeStruct((NWORKERS, TROWS, 128), jnp.float32),
            jax.ShapeDtypeStruct((NWORKERS, TROWS, 128), jnp.float32),
        ),
        scratch_types=[
            pltpu.VMEM((piece,), jnp.float32),
            pltpu.VMEM((piece,), jnp.float32),
            pltpu.VMEM((TROWS, 128), jnp.float32),
            pltpu.VMEM((TROWS, 128), jnp.float32),
            pltpu.SemaphoreType.DMA,
            pltpu.SemaphoreType.DMA,
        ],
    )(_sc_hist_body)


def _merge_body(cnt_ref, sum_ref, cnt_out, sum_out):
    cnt_out[...] = jnp.sum(cnt_ref[...], axis=0)
    sum_out[...] = jnp.sum(sum_ref[...], axis=0)


def _premerge(cnt, sm):
    spec3 = pl.BlockSpec((NWORKERS, TROWS, 128), lambda: (0, 0, 0))
    spec2 = pl.BlockSpec((TROWS, 128), lambda: (0, 0))
    return pl.pallas_call(
        _merge_body,
        in_specs=[spec3, spec3],
        out_specs=(spec2, spec2),
        out_shape=(jax.ShapeDtypeStruct((TROWS, 128), jnp.float32),
                   jax.ShapeDtypeStruct((TROWS, 128), jnp.float32)),
    )(cnt, sm)


def _select_body(cnt0_ref, sum0_ref, cnt1_ref, sum1_ref, out_ref):
    cnt = cnt0_ref[...] + jnp.sum(cnt1_ref[...], axis=0)
    sm = sum0_ref[...] + jnp.sum(sum1_ref[...], axis=0)

    r_i = lax.broadcasted_iota(jnp.int32, (TROWS, 128), 0)
    c_i = lax.broadcasted_iota(jnp.int32, (TROWS, 128), 1)
    bin_idx = lax.shift_right_logical(r_i * 128 + c_i, 4)

    kf = jnp.float32(K_KEEP)

    def n_gt(x):
        return jnp.sum(jnp.where(bin_idx > x, cnt, 0.0))

    def bis(_, lohi):
        lo, hi = lohi
        mid = (lo + hi) // 2
        below = n_gt(mid) < kf
        return jnp.where(below, lo, mid), jnp.where(below, mid, hi)

    lo, hi = lax.fori_loop(0, 12, bis, (jnp.int32(-1), jnp.int32(NBINS - 1)))
    b_sel = hi

    count_above = n_gt(b_sel)
    sum_above = jnp.sum(jnp.where(bin_idx > b_sel, sm, 0.0))
    c_b = jnp.sum(jnp.where(bin_idx == b_sel, cnt, 0.0))
    s_b = jnp.sum(jnp.where(bin_idx == b_sel, sm, 0.0))
    r = kf - count_above

    lo_f_vec = lax.bitcast_convert_type(
        lax.shift_left(bin_idx, SHIFT), jnp.float32)
    lo_f = jnp.sum(jnp.where(bin_idx == b_sel, lo_f_vec, 0.0)) / NLANE
    hi_f = jnp.sum(jnp.where(bin_idx == b_sel + 1, lo_f_vec, 0.0)) / NLANE
    width = hi_f - lo_f

    mhat = (s_b / c_b - lo_f) / width
    bq = jnp.clip(12.0 * mhat - 6.0, -2.0, 2.0)
    aq = 1.0 - bq / 2.0
    tq = r / c_b
    small_b = jnp.abs(bq) < 1e-3
    b_safe = jnp.where(small_b, 1.0, bq)
    disc = jnp.maximum(aq * aq + 2.0 * b_safe * (aq + b_safe / 2.0 - tq), 0.0)
    sdisc = jnp.max(jnp.sqrt(jnp.full((8, 128), disc, jnp.float32)))
    q_quad = (sdisc - aq) / b_safe
    q_lin = 1.0 - tq / aq
    q = jnp.clip(jnp.where(small_b, q_lin, q_quad), 0.0, 1.0)
    iu = aq * (1.0 - q * q) / 2.0 + bq * (1.0 - q * q * q) / 3.0
    top_r = c_b * width * iu + r * lo_f
    out_ref[0, 0] = (sum_above + top_r) / kf


def _select(cnt0m, sm0m, cnt1, sm1):
    spec3 = pl.BlockSpec((NWORKERS, TROWS, 128), lambda: (0, 0, 0))
    spec2 = pl.BlockSpec((TROWS, 128), lambda: (0, 0))
    return pl.pallas_call(
        _select_body,
        in_specs=[spec2, spec2, spec3, spec3],
        out_specs=pl.BlockSpec(memory_space=pltpu.SMEM),
        out_shape=jax.ShapeDtypeStruct((1, 1), jnp.float32),
    )(cnt0m, sm0m, cnt1, sm1)


def kernel(inputs, targets):
    sc_hist = _make_sc_histograms(N0)
    losses0 = _pixel_losses(inputs, targets, 0, SPLIT)
    cnt0, sm0 = sc_hist(losses0.reshape(N0))
    losses1 = _pixel_losses(inputs, targets, SPLIT, B - SPLIT)
    if N1 == N0:
        cnt1, sm1 = sc_hist(losses1.reshape(N1))
    else:
        cnt1, sm1 = _make_sc_histograms(N1)(losses1.reshape(N1))
    cnt0m, sm0m = _premerge(cnt0, sm0)
    out = _select(cnt0m, sm0m, cnt1, sm1)
    return out[0, 0]


# speedup vs baseline: 1.0283x; 1.0105x over previous
import functools

import jax
import jax.numpy as jnp
from jax import lax
from jax.experimental import pallas as pl
from jax.experimental.pallas import tpu as pltpu
from jax.experimental.pallas import tpu_sc as plsc

THRESH_IGNORE = 255
K_KEEP = 100000

B, C, H, W = 8, 19, 512, 512
N = B * H * W
SPLIT = 6
N0 = SPLIT * H * W
N1 = N - N0

SHIFT = 21
NBINS = 1024
NLANE = 16
TROWS = NBINS * NLANE // 128

NWORKERS = 32

ROWS = 256
OROWS = ROWS * W // 128


def _loss_body(x_ref, t_ref, out_ref, rows):
    x = x_ref[0]
    t = t_ref[0]
    m = jnp.max(x, axis=0)
    s = jnp.sum(jnp.exp(x - m[None]), axis=0)
    xt = jnp.zeros_like(m)
    for c in range(C):
        xt = jnp.where(t == c, x[c], xt)
    nll = jnp.log(s) + m - xt
    nll = jnp.where(t == THRESH_IGNORE, 0.0, nll)
    for j in range(W // 128):
        out_ref[pl.ds(j * rows, rows), :] = nll[:, j * 128:(j + 1) * 128]


def _pixel_losses(inputs, targets, b0, bn, rows=ROWS):
    grid = (bn, H // rows)
    nb = H // rows
    orows = rows * W // 128

    def body(x_ref, t_ref, out_ref):
        _loss_body(x_ref, t_ref, out_ref, rows)

    return pl.pallas_call(
        body,
        grid=grid,
        in_specs=[
            pl.BlockSpec((1, C, rows, W), lambda b, h: (b + b0, 0, h, 0)),
            pl.BlockSpec((1, rows, W), lambda b, h: (b + b0, h, 0)),
        ],
        out_specs=pl.BlockSpec((orows, 128), lambda b, h: (b * nb + h, 0)),
        out_shape=jax.ShapeDtypeStruct((bn * nb * orows, 128), jnp.float32),
    )(inputs, targets)


def _make_sc_histograms(n_elems):
    chunk = n_elems // NWORKERS
    npiece = 2
    piece = chunk // npiece

    def _sc_hist_body(loss_hbm, cnt_out, sum_out,
                      buf0, buf1, cnt_tab, sum_tab, sem0, sem1):
        cid = lax.axis_index("c")
        sid = lax.axis_index("s")
        wid = sid * 2 + cid
        base = wid * chunk

        zeros = jnp.zeros((NLANE,), jnp.float32)

        @plsc.parallel_loop(0, TROWS * 8, unroll=8)
        def zero_body(i):
            r = lax.shift_right_logical(i, 3)
            c0 = (i & 7) * NLANE
            cnt_tab[r, pl.ds(c0, NLANE)] = zeros
            sum_tab[r, pl.ds(c0, NLANE)] = zeros

        lane = lax.iota(jnp.int32, NLANE)
        ones = jnp.ones((NLANE,), jnp.float32)
        bufs = (buf0, buf1)
        sems = (sem0, sem1)

        copies = [None] * npiece
        copies[0] = pltpu.make_async_copy(
            loss_hbm.at[pl.ds(base, piece)], buf0, sem0)
        copies[0].start()
        for p in range(npiece):
            if p + 1 < npiece:
                copies[p + 1] = pltpu.make_async_copy(
                    loss_hbm.at[pl.ds(base + (p + 1) * piece, piece)],
                    bufs[(p + 1) % 2], sems[(p + 1) % 2])
                copies[p + 1].start()
            copies[p].wait()
            buf = bufs[p % 2]

            @plsc.parallel_loop(0, piece // NLANE, unroll=8)
            def hist_body(i):
                v = buf[pl.ds(i * NLANE, NLANE)]
                bits = plsc.bitcast(v, jnp.int32)
                idx = lax.shift_right_logical(bits, SHIFT) * NLANE + lane
                row = lax.shift_right_logical(idx, 7)
                col = idx & 127
                plsc.addupdate_scatter(cnt_tab, [row, col], ones)
                plsc.addupdate_scatter(sum_tab, [row, col], v)

        pltpu.sync_copy(cnt_tab, cnt_out.at[wid])
        pltpu.sync_copy(sum_tab, sum_out.at[wid])

    mesh = plsc.VectorSubcoreMesh(core_axis_name="c", subcore_axis_name="s")
    return functools.partial(
        pl.kernel,
        mesh=mesh,
        compiler_params=pltpu.CompilerParams(needs_layout_passes=False),
        out_type=(
            jax.ShapeDtypeStruct((NWORKERS, TROWS, 128), jnp.float32),
            jax.ShapeDtypeStruct((NWORKERS, TROWS, 128), jnp.float32),
        ),
        scratch_types=[
            pltpu.VMEM((piece,), jnp.float32),
            pltpu.VMEM((piece,), jnp.float32),
            pltpu.VMEM((TROWS, 128), jnp.float32),
            pltpu.VMEM((TROWS, 128), jnp.float32),
            pltpu.SemaphoreType.DMA,
            pltpu.SemaphoreType.DMA,
        ],
    )(_sc_hist_body)


def _merge_body(cnt_ref, sum_ref, cnt_out, sum_out):
    cnt_out[...] = jnp.sum(cnt_ref[...], axis=0)
    sum_out[...] = jnp.sum(sum_ref[...], axis=0)


def _premerge(cnt, sm):
    spec3 = pl.BlockSpec((NWORKERS, TROWS, 128), lambda: (0, 0, 0))
    spec2 = pl.BlockSpec((TROWS, 128), lambda: (0, 0))
    return pl.pallas_call(
        _merge_body,
        in_specs=[spec3, spec3],
        out_specs=(spec2, spec2),
        out_shape=(jax.ShapeDtypeStruct((TROWS, 128), jnp.float32),
                   jax.ShapeDtypeStruct((TROWS, 128), jnp.float32)),
    )(cnt, sm)


def _select_body(cnt0_ref, sum0_ref, cnt1_ref, sum1_ref, out_ref):
    cnt = cnt0_ref[...] + jnp.sum(cnt1_ref[...], axis=0)
    sm = sum0_ref[...] + jnp.sum(sum1_ref[...], axis=0)

    r_i = lax.broadcasted_iota(jnp.int32, (TROWS, 128), 0)
    c_i = lax.broadcasted_iota(jnp.int32, (TROWS, 128), 1)
    bin_idx = lax.shift_right_logical(r_i * 128 + c_i, 4)

    kf = jnp.float32(K_KEEP)

    def n_gt(x):
        return jnp.sum(jnp.where(bin_idx > x, cnt, 0.0))

    def bis(_, lohi):
        lo, hi = lohi
        mid = (lo + hi) // 2
        below = n_gt(mid) < kf
        return jnp.where(below, lo, mid), jnp.where(below, mid, hi)

    lo, hi = lax.fori_loop(0, 12, bis, (jnp.int32(-1), jnp.int32(NBINS - 1)))
    b_sel = hi

    count_above = n_gt(b_sel)
    sum_above = jnp.sum(jnp.where(bin_idx > b_sel, sm, 0.0))
    c_b = jnp.sum(jnp.where(bin_idx == b_sel, cnt, 0.0))
    s_b = jnp.sum(jnp.where(bin_idx == b_sel, sm, 0.0))
    r = kf - count_above

    lo_f_vec = lax.bitcast_convert_type(
        lax.shift_left(bin_idx, SHIFT), jnp.float32)
    lo_f = jnp.sum(jnp.where(bin_idx == b_sel, lo_f_vec, 0.0)) / NLANE
    hi_f = jnp.sum(jnp.where(bin_idx == b_sel + 1, lo_f_vec, 0.0)) / NLANE
    width = hi_f - lo_f

    mhat = (s_b / c_b - lo_f) / width
    bq = jnp.clip(12.0 * mhat - 6.0, -2.0, 2.0)
    aq = 1.0 - bq / 2.0
    tq = r / c_b
    small_b = jnp.abs(bq) < 1e-3
    b_safe = jnp.where(small_b, 1.0, bq)
    disc = jnp.maximum(aq * aq + 2.0 * b_safe * (aq + b_safe / 2.0 - tq), 0.0)
    sdisc = jnp.max(jnp.sqrt(jnp.full((8, 128), disc, jnp.float32)))
    q_quad = (sdisc - aq) / b_safe
    q_lin = 1.0 - tq / aq
    q = jnp.clip(jnp.where(small_b, q_lin, q_quad), 0.0, 1.0)
    iu = aq * (1.0 - q * q) / 2.0 + bq * (1.0 - q * q * q) / 3.0
    top_r = c_b * width * iu + r * lo_f
    out_ref[0, 0] = (sum_above + top_r) / kf


def _select(cnt0m, sm0m, cnt1, sm1):
    spec3 = pl.BlockSpec((NWORKERS, TROWS, 128), lambda: (0, 0, 0))
    spec2 = pl.BlockSpec((TROWS, 128), lambda: (0, 0))
    return pl.pallas_call(
        _select_body,
        in_specs=[spec2, spec2, spec3, spec3],
        out_specs=pl.BlockSpec(memory_space=pltpu.SMEM),
        out_shape=jax.ShapeDtypeStruct((1, 1), jnp.float32),
    )(cnt0m, sm0m, cnt1, sm1)


def kernel(inputs, targets):
    sc_hist = _make_sc_histograms(N0)
    losses0 = _pixel_losses(inputs, targets, 0, SPLIT)
    cnt0, sm0 = sc_hist(losses0.reshape(N0))
    losses1 = _pixel_losses(inputs, targets, SPLIT, B - SPLIT, rows=128)
    if N1 == N0:
        cnt1, sm1 = sc_hist(losses1.reshape(N1))
    else:
        cnt1, sm1 = _make_sc_histograms(N1)(losses1.reshape(N1))
    cnt0m, sm0m = _premerge(cnt0, sm0)
    out = _select(cnt0m, sm0m, cnt1, sm1)
    return out[0, 0]


# speedup vs baseline: 1.0343x; 1.0058x over previous
import functools

import jax
import jax.numpy as jnp
from jax import lax
from jax.experimental import pallas as pl
from jax.experimental.pallas import tpu as pltpu
from jax.experimental.pallas import tpu_sc as plsc

THRESH_IGNORE = 255
K_KEEP = 100000

B, C, H, W = 8, 19, 512, 512
N = B * H * W
SPLIT = 6
N0 = SPLIT * H * W
N1 = N - N0

SHIFT = 21
NBINS = 1024
NLANE = 16
TROWS = NBINS * NLANE // 128

NWORKERS = 32

ROWS = 256
OROWS = ROWS * W // 128


def _loss_body(x_ref, t_ref, out_ref, rows):
    x = x_ref[0]
    t = t_ref[0]
    m = jnp.max(x, axis=0)
    s = jnp.sum(jnp.exp(x - m[None]), axis=0)
    xt = jnp.zeros_like(m)
    for c in range(C):
        xt = jnp.where(t == c, x[c], xt)
    nll = jnp.log(s) + m - xt
    nll = jnp.where(t == THRESH_IGNORE, 0.0, nll)
    for j in range(W // 128):
        out_ref[pl.ds(j * rows, rows), :] = nll[:, j * 128:(j + 1) * 128]


def _pixel_losses(inputs, targets, b0, bn, rows=ROWS):
    grid = (bn, H // rows)
    nb = H // rows
    orows = rows * W // 128

    def body(x_ref, t_ref, out_ref):
        _loss_body(x_ref, t_ref, out_ref, rows)

    return pl.pallas_call(
        body,
        grid=grid,
        in_specs=[
            pl.BlockSpec((1, C, rows, W), lambda b, h: (b + b0, 0, h, 0)),
            pl.BlockSpec((1, rows, W), lambda b, h: (b + b0, h, 0)),
        ],
        out_specs=pl.BlockSpec((orows, 128), lambda b, h: (b * nb + h, 0)),
        out_shape=jax.ShapeDtypeStruct((bn * nb * orows, 128), jnp.float32),
    )(inputs, targets)


def _make_sc_histograms(n_elems):
    chunk = n_elems // NWORKERS
    npiece = 2
    piece = chunk // npiece

    def _sc_hist_body(loss_hbm, cnt_out, sum_out,
                      buf0, buf1, cnt_tab, sum_tab, sem0, sem1):
        cid = lax.axis_index("c")
        sid = lax.axis_index("s")
        wid = sid * 2 + cid
        base = wid * chunk

        lane = lax.iota(jnp.int32, NLANE)
        ones = jnp.ones((NLANE,), jnp.float32)
        zeros = jnp.zeros((NLANE,), jnp.float32)
        bufs = (buf0, buf1)
        sems = (sem0, sem1)

        copies = [None] * npiece
        copies[0] = pltpu.make_async_copy(
            loss_hbm.at[pl.ds(base, piece)], buf0, sem0)
        copies[0].start()

        @plsc.parallel_loop(0, TROWS * 8, unroll=8)
        def zero_body(i):
            r = lax.shift_right_logical(i, 3)
            c0 = (i & 7) * NLANE
            cnt_tab[r, pl.ds(c0, NLANE)] = zeros
            sum_tab[r, pl.ds(c0, NLANE)] = zeros
        for p in range(npiece):
            if p + 1 < npiece:
                copies[p + 1] = pltpu.make_async_copy(
                    loss_hbm.at[pl.ds(base + (p + 1) * piece, piece)],
                    bufs[(p + 1) % 2], sems[(p + 1) % 2])
                copies[p + 1].start()
            copies[p].wait()
            buf = bufs[p % 2]

            @plsc.parallel_loop(0, piece // NLANE, unroll=8)
            def hist_body(i):
                v = buf[pl.ds(i * NLANE, NLANE)]
                bits = plsc.bitcast(v, jnp.int32)
                idx = lax.shift_right_logical(bits, SHIFT) * NLANE + lane
                row = lax.shift_right_logical(idx, 7)
                col = idx & 127
                plsc.addupdate_scatter(cnt_tab, [row, col], ones)
                plsc.addupdate_scatter(sum_tab, [row, col], v)

        pltpu.sync_copy(cnt_tab, cnt_out.at[wid])
        pltpu.sync_copy(sum_tab, sum_out.at[wid])

    mesh = plsc.VectorSubcoreMesh(core_axis_name="c", subcore_axis_name="s")
    return functools.partial(
        pl.kernel,
        mesh=mesh,
        compiler_params=pltpu.CompilerParams(needs_layout_passes=False),
        out_type=(
            jax.ShapeDtypeStruct((NWORKERS, TROWS, 128), jnp.float32),
            jax.ShapeDtypeStruct((NWORKERS, TROWS, 128), jnp.float32),
        ),
        scratch_types=[
            pltpu.VMEM((piece,), jnp.float32),
            pltpu.VMEM((piece,), jnp.float32),
            pltpu.VMEM((TROWS, 128), jnp.float32),
            pltpu.VMEM((TROWS, 128), jnp.float32),
            pltpu.SemaphoreType.DMA,
            pltpu.SemaphoreType.DMA,
        ],
    )(_sc_hist_body)


def _merge_body(cnt_ref, sum_ref, cnt_out, sum_out):
    cnt_out[...] = jnp.sum(cnt_ref[...], axis=0)
    sum_out[...] = jnp.sum(sum_ref[...], axis=0)


def _premerge(cnt, sm):
    spec3 = pl.BlockSpec((NWORKERS, TROWS, 128), lambda: (0, 0, 0))
    spec2 = pl.BlockSpec((TROWS, 128), lambda: (0, 0))
    return pl.pallas_call(
        _merge_body,
        in_specs=[spec3, spec3],
        out_specs=(spec2, spec2),
        out_shape=(jax.ShapeDtypeStruct((TROWS, 128), jnp.float32),
                   jax.ShapeDtypeStruct((TROWS, 128), jnp.float32)),
    )(cnt, sm)


def _select_body(cnt0_ref, sum0_ref, cnt1_ref, sum1_ref, out_ref):
    cnt = cnt0_ref[...] + jnp.sum(cnt1_ref[...], axis=0)
    sm = sum0_ref[...] + jnp.sum(sum1_ref[...], axis=0)

    r_i = lax.broadcasted_iota(jnp.int32, (TROWS, 128), 0)
    c_i = lax.broadcasted_iota(jnp.int32, (TROWS, 128), 1)
    bin_idx = lax.shift_right_logical(r_i * 128 + c_i, 4)

    kf = jnp.float32(K_KEEP)

    def n_gt(x):
        return jnp.sum(jnp.where(bin_idx > x, cnt, 0.0))

    def bis(_, lohi):
        lo, hi = lohi
        mid = (lo + hi) // 2
        below = n_gt(mid) < kf
        return jnp.where(below, lo, mid), jnp.where(below, mid, hi)

    lo, hi = lax.fori_loop(0, 12, bis, (jnp.int32(-1), jnp.int32(NBINS - 1)))
    b_sel = hi

    count_above = n_gt(b_sel)
    sum_above = jnp.sum(jnp.where(bin_idx > b_sel, sm, 0.0))
    c_b = jnp.sum(jnp.where(bin_idx == b_sel, cnt, 0.0))
    s_b = jnp.sum(jnp.where(bin_idx == b_sel, sm, 0.0))
    r = kf - count_above

    lo_f_vec = lax.bitcast_convert_type(
        lax.shift_left(bin_idx, SHIFT), jnp.float32)
    lo_f = jnp.sum(jnp.where(bin_idx == b_sel, lo_f_vec, 0.0)) / NLANE
    hi_f = jnp.sum(jnp.where(bin_idx == b_sel + 1, lo_f_vec, 0.0)) / NLANE
    width = hi_f - lo_f

    mhat = (s_b / c_b - lo_f) / width
    bq = jnp.clip(12.0 * mhat - 6.0, -2.0, 2.0)
    aq = 1.0 - bq / 2.0
    tq = r / c_b
    small_b = jnp.abs(bq) < 1e-3
    b_safe = jnp.where(small_b, 1.0, bq)
    disc = jnp.maximum(aq * aq + 2.0 * b_safe * (aq + b_safe / 2.0 - tq), 0.0)
    sdisc = jnp.max(jnp.sqrt(jnp.full((8, 128), disc, jnp.float32)))
    q_quad = (sdisc - aq) / b_safe
    q_lin = 1.0 - tq / aq
    q = jnp.clip(jnp.where(small_b, q_lin, q_quad), 0.0, 1.0)
    iu = aq * (1.0 - q * q) / 2.0 + bq * (1.0 - q * q * q) / 3.0
    top_r = c_b * width * iu + r * lo_f
    out_ref[0, 0] = (sum_above + top_r) / kf


def _select(cnt0m, sm0m, cnt1, sm1):
    spec3 = pl.BlockSpec((NWORKERS, TROWS, 128), lambda: (0, 0, 0))
    spec2 = pl.BlockSpec((TROWS, 128), lambda: (0, 0))
    return pl.pallas_call(
        _select_body,
        in_specs=[spec2, spec2, spec3, spec3],
        out_specs=pl.BlockSpec(memory_space=pltpu.SMEM),
        out_shape=jax.ShapeDtypeStruct((1, 1), jnp.float32),
    )(cnt0m, sm0m, cnt1, sm1)


def kernel(inputs, targets):
    sc_hist = _make_sc_histograms(N0)
    losses0 = _pixel_losses(inputs, targets, 0, SPLIT)
    cnt0, sm0 = sc_hist(losses0.reshape(N0))
    losses1 = _pixel_losses(inputs, targets, SPLIT, B - SPLIT, rows=128)
    if N1 == N0:
        cnt1, sm1 = sc_hist(losses1.reshape(N1))
    else:
        cnt1, sm1 = _make_sc_histograms(N1)(losses1.reshape(N1))
    cnt0m, sm0m = _premerge(cnt0, sm0)
    out = _select(cnt0m, sm0m, cnt1, sm1)
    return out[0, 0]
